# baseline (device time: 83824 ns/iter reference)
import jax
import jax.numpy as jnp
from jax import lax
from jax.experimental import pallas as pl
from jax.experimental.pallas import tpu as pltpu

N_DEV = 4
H = N_DEV - 1


def kernel(x):
    m_per, n = x.shape
    half = m_per // 2

    def body(x_ref, out_ref, comm_f, comm_b,
             fs_sems, fr_sems, bs_sems, br_sems, cp_sems):
        my_pos = lax.axis_index("i")
        left = (my_pos - 1) % N_DEV
        right = (my_pos + 1) % N_DEV

        barrier_sem = pltpu.get_barrier_semaphore()
        for nbr in [left, right]:
            pl.semaphore_signal(
                barrier_sem, inc=1,
                device_id=(nbr,), device_id_type=pl.DeviceIdType.MESH,
            )
        pl.semaphore_wait(barrier_sem, 2)

        def remote(src, h, fwd):
            return pltpu.make_async_remote_copy(
                src_ref=src,
                dst_ref=(comm_f if fwd else comm_b).at[h],
                send_sem=(fs_sems if fwd else bs_sems).at[h],
                recv_sem=(fr_sems if fwd else br_sems).at[h],
                device_id=(right if fwd else left,),
                device_id_type=pl.DeviceIdType.MESH,
            )

        f = remote(x_ref.at[pl.ds(0, half)], 0, True)
        b = remote(x_ref.at[pl.ds(half, half)], 0, False)
        f.start()
        b.start()

        own = pltpu.make_async_copy(
            x_ref, out_ref.at[pl.ds(my_pos * m_per, m_per)], cp_sems.at[2 * H]
        )
        own.start()

        copies = [own]
        rdmas = [f, b]
        for h in range(H):
            o_f = (my_pos - 1 - h) % N_DEV
            o_b = (my_pos + 1 + h) % N_DEV
            f.wait_recv()
            if h + 1 < H:
                f_next = remote(comm_f.at[h], h + 1, True)
                f_next.start()
                rdmas.append(f_next)
            cp = pltpu.make_async_copy(
                comm_f.at[h], out_ref.at[pl.ds(o_f * m_per, half)],
                cp_sems.at[h],
            )
            cp.start()
            copies.append(cp)

            b.wait_recv()
            if h + 1 < H:
                b_next = remote(comm_b.at[h], h + 1, False)
                b_next.start()
                rdmas.append(b_next)
                f, b = f_next, b_next
            cp = pltpu.make_async_copy(
                comm_b.at[h], out_ref.at[pl.ds(o_b * m_per + half, half)],
                cp_sems.at[H + h],
            )
            cp.start()
            copies.append(cp)

        for cp in copies:
            cp.wait()
        for r in rdmas:
            r.wait_send()

    return pl.pallas_call(
        body,
        out_shape=jax.ShapeDtypeStruct((N_DEV * m_per, n), x.dtype),
        in_specs=[pl.BlockSpec(memory_space=pltpu.VMEM)],
        out_specs=pl.BlockSpec(memory_space=pl.ANY),
        scratch_shapes=[
            pltpu.VMEM((H, half, n), x.dtype),
            pltpu.VMEM((H, half, n), x.dtype),
            pltpu.SemaphoreType.DMA((H,)),
            pltpu.SemaphoreType.DMA((H,)),
            pltpu.SemaphoreType.DMA((H,)),
            pltpu.SemaphoreType.DMA((H,)),
            pltpu.SemaphoreType.DMA((2 * H + 1,)),
        ],
        compiler_params=pltpu.CompilerParams(collective_id=0),
    )(x)


# device time: 80861 ns/iter; 1.0366x vs baseline; 1.0366x over previous
import jax
import jax.numpy as jnp
from jax import lax
from jax.experimental import pallas as pl
from jax.experimental.pallas import tpu as pltpu

N_DEV = 4
H = N_DEV - 1
S = 2


def kernel(x):
    m_per, n = x.shape
    half = m_per // 2
    seg = half // S

    def body(x_ref, out_ref, comm_f, comm_b,
             fs_sems, fr_sems, bs_sems, br_sems, cp_sems, own_sem):
        my_pos = lax.axis_index("i")
        left = (my_pos - 1) % N_DEV
        right = (my_pos + 1) % N_DEV

        barrier_sem = pltpu.get_barrier_semaphore()
        for nbr in [left, right]:
            pl.semaphore_signal(
                barrier_sem, inc=1,
                device_id=(nbr,), device_id_type=pl.DeviceIdType.MESH,
            )
        pl.semaphore_wait(barrier_sem, 2)

        def remote(src, h, s, fwd):
            return pltpu.make_async_remote_copy(
                src_ref=src,
                dst_ref=(comm_f if fwd else comm_b).at[h, pl.ds(s * seg, seg)],
                send_sem=(fs_sems if fwd else bs_sems).at[h, s],
                recv_sem=(fr_sems if fwd else br_sems).at[h, s],
                device_id=(right if fwd else left,),
                device_id_type=pl.DeviceIdType.MESH,
            )

        f_prev, b_prev = [], []
        for s in range(S):
            r = remote(x_ref.at[pl.ds(s * seg, seg)], 0, s, True)
            r.start()
            f_prev.append(r)
            r = remote(x_ref.at[pl.ds(half + s * seg, seg)], 0, s, False)
            r.start()
            b_prev.append(r)

        own = pltpu.make_async_copy(
            x_ref, out_ref.at[pl.ds(my_pos * m_per, m_per)], own_sem
        )
        own.start()

        rdmas = list(f_prev) + list(b_prev)
        copies = []
        for h in range(H):
            o_f = (my_pos - 1 - h) % N_DEV
            o_b = (my_pos + 1 + h) % N_DEV
            f_cur, b_cur = [], []
            for s in range(S):
                f_prev[s].wait_recv()
                if h + 1 < H:
                    r = remote(comm_f.at[h, pl.ds(s * seg, seg)], h + 1, s, True)
                    r.start()
                    f_cur.append(r)
                    rdmas.append(r)
                cp = pltpu.make_async_copy(
                    comm_f.at[h, pl.ds(s * seg, seg)],
                    out_ref.at[pl.ds(o_f * m_per + s * seg, seg)],
                    cp_sems.at[0, h, s],
                )
                cp.start()
                copies.append(cp)

                b_prev[s].wait_recv()
                if h + 1 < H:
                    r = remote(comm_b.at[h, pl.ds(s * seg, seg)], h + 1, s, False)
                    r.start()
                    b_cur.append(r)
                    rdmas.append(r)
                cp = pltpu.make_async_copy(
                    comm_b.at[h, pl.ds(s * seg, seg)],
                    out_ref.at[pl.ds(o_b * m_per + half + s * seg, seg)],
                    cp_sems.at[1, h, s],
                )
                cp.start()
                copies.append(cp)
            f_prev, b_prev = f_cur, b_cur

        own.wait()
        for cp in copies:
            cp.wait()
        for r in rdmas:
            r.wait_send()

    return pl.pallas_call(
        body,
        out_shape=jax.ShapeDtypeStruct((N_DEV * m_per, n), x.dtype),
        in_specs=[pl.BlockSpec(memory_space=pl.ANY)],
        out_specs=pl.BlockSpec(memory_space=pl.ANY),
        scratch_shapes=[
            pltpu.VMEM((H, half, n), x.dtype),
            pltpu.VMEM((H, half, n), x.dtype),
            pltpu.SemaphoreType.DMA((H, S)),
            pltpu.SemaphoreType.DMA((H, S)),
            pltpu.SemaphoreType.DMA((H, S)),
            pltpu.SemaphoreType.DMA((H, S)),
            pltpu.SemaphoreType.DMA((2, H, S)),
            pltpu.SemaphoreType.DMA,
        ],
        compiler_params=pltpu.CompilerParams(collective_id=0),
    )(x)


# device time: 79038 ns/iter; 1.0606x vs baseline; 1.0231x over previous
import jax
import jax.numpy as jnp
from jax import lax
from jax.experimental import pallas as pl
from jax.experimental.pallas import tpu as pltpu

N_DEV = 4
H = N_DEV - 1
S = 2


def kernel(x):
    m_per, n = x.shape
    half = m_per // 2
    seg = half // S

    def body(x_ref, out_ref, comm_f, comm_b,
             fs_sems, fr_sems, bs_sems, br_sems, cp_sems, own_sem):
        my_pos = lax.axis_index("i")
        left = (my_pos - 1) % N_DEV
        right = (my_pos + 1) % N_DEV

        barrier_sem = pltpu.get_barrier_semaphore()
        for nbr in [left, right]:
            pl.semaphore_signal(
                barrier_sem, inc=1,
                device_id=(nbr,), device_id_type=pl.DeviceIdType.MESH,
            )
        pl.semaphore_wait(barrier_sem, 2)

        def remote(src, h, s, fwd):
            return pltpu.make_async_remote_copy(
                src_ref=src,
                dst_ref=(comm_f if fwd else comm_b).at[h, pl.ds(s * seg, seg)],
                send_sem=(fs_sems if fwd else bs_sems).at[h, s],
                recv_sem=(fr_sems if fwd else br_sems).at[h, s],
                device_id=(right if fwd else left,),
                device_id_type=pl.DeviceIdType.MESH,
            )

        f_prev, b_prev = [], []
        for s in range(S):
            r = remote(x_ref.at[pl.ds(s * seg, seg)], 0, s, True)
            r.start()
            f_prev.append(r)
            r = remote(x_ref.at[pl.ds(half + s * seg, seg)], 0, s, False)
            r.start()
            b_prev.append(r)

        own = pltpu.make_async_copy(
            x_ref, out_ref.at[pl.ds(my_pos * m_per, m_per)], own_sem
        )
        own.start()

        rdmas = list(f_prev) + list(b_prev)
        copies = []
        for h in range(H):
            o_f = (my_pos - 1 - h) % N_DEV
            o_b = (my_pos + 1 + h) % N_DEV
            f_cur, b_cur = [], []
            for s in range(S):
                f_prev[s].wait_recv()
                if h + 1 < H:
                    r = remote(comm_f.at[h, pl.ds(s * seg, seg)], h + 1, s, True)
                    r.start()
                    f_cur.append(r)
                    rdmas.append(r)
                cp = pltpu.make_async_copy(
                    comm_f.at[h, pl.ds(s * seg, seg)],
                    out_ref.at[pl.ds(o_f * m_per + s * seg, seg)],
                    cp_sems.at[0, h, s],
                )
                cp.start()
                copies.append(cp)

                b_prev[s].wait_recv()
                if h + 1 < H:
                    r = remote(comm_b.at[h, pl.ds(s * seg, seg)], h + 1, s, False)
                    r.start()
                    b_cur.append(r)
                    rdmas.append(r)
                cp = pltpu.make_async_copy(
                    comm_b.at[h, pl.ds(s * seg, seg)],
                    out_ref.at[pl.ds(o_b * m_per + half + s * seg, seg)],
                    cp_sems.at[1, h, s],
                )
                cp.start()
                copies.append(cp)
            f_prev, b_prev = f_cur, b_cur

        own.wait()
        for cp in copies:
            cp.wait()
        for r in rdmas:
            r.wait_send()

    x = pltpu.with_memory_space_constraint(x, pltpu.MemorySpace.HBM)
    return pl.pallas_call(
        body,
        out_shape=jax.ShapeDtypeStruct((N_DEV * m_per, n), x.dtype),
        in_specs=[pl.BlockSpec(memory_space=pltpu.MemorySpace.HBM)],
        out_specs=pl.BlockSpec(memory_space=pltpu.MemorySpace.HBM),
        scratch_shapes=[
            pltpu.VMEM((H, half, n), x.dtype),
            pltpu.VMEM((H, half, n), x.dtype),
            pltpu.SemaphoreType.DMA((H, S)),
            pltpu.SemaphoreType.DMA((H, S)),
            pltpu.SemaphoreType.DMA((H, S)),
            pltpu.SemaphoreType.DMA((H, S)),
            pltpu.SemaphoreType.DMA((2, H, S)),
            pltpu.SemaphoreType.DMA,
        ],
        compiler_params=pltpu.CompilerParams(collective_id=0),
    )(x)


# device time: 79005 ns/iter; 1.0610x vs baseline; 1.0004x over previous
import jax
import jax.numpy as jnp
from jax import lax
from jax.experimental import pallas as pl
from jax.experimental.pallas import tpu as pltpu

N_DEV = 4


def kernel(x):
    m_per, n = x.shape
    half = m_per // 2

    def body(x_ref, out_ref, comm_f, comm_b,
             s_sems, r_sems, cp_sems, own_sem):
        my_pos = lax.axis_index("i")
        left = (my_pos - 1) % N_DEV
        right = (my_pos + 1) % N_DEV

        barrier_sem = pltpu.get_barrier_semaphore()
        for nbr in [left, right]:
            pl.semaphore_signal(
                barrier_sem, inc=1,
                device_id=(nbr,), device_id_type=pl.DeviceIdType.MESH,
            )
        pl.semaphore_wait(barrier_sem, 2)

        def remote(src, comm, slot, target, i):
            return pltpu.make_async_remote_copy(
                src_ref=src,
                dst_ref=comm.at[slot],
                send_sem=s_sems.at[i],
                recv_sem=r_sems.at[i],
                device_id=(target,),
                device_id_type=pl.DeviceIdType.MESH,
            )

        f0 = remote(x_ref.at[pl.ds(0, half)], comm_f, 0, right, 0)
        b0 = remote(x_ref.at[pl.ds(half, half)], comm_b, 0, left, 1)
        f1 = remote(x_ref.at[pl.ds(half, half)], comm_f, 1, right, 2)
        b1 = remote(x_ref.at[pl.ds(0, half)], comm_b, 1, left, 3)
        f0.start()
        b0.start()
        f1.start()
        b1.start()

        own = pltpu.make_async_copy(
            x_ref, out_ref.at[pl.ds(my_pos * m_per, m_per)], own_sem
        )
        own.start()

        def copy_out(comm, slot, row, i):
            cp = pltpu.make_async_copy(
                comm.at[slot], out_ref.at[pl.ds(row, half)], cp_sems.at[i]
            )
            cp.start()
            return cp

        f0.wait_recv()
        f2 = remote(comm_f.at[0], comm_f, 2, right, 4)
        f2.start()
        cps = [copy_out(comm_f, 0, left * m_per, 0)]
        b0.wait_recv()
        b2 = remote(comm_b.at[0], comm_b, 2, left, 5)
        b2.start()
        cps.append(copy_out(comm_b, 0, right * m_per + half, 1))

        f1.wait_recv()
        cps.append(copy_out(comm_f, 1, left * m_per + half, 2))
        b1.wait_recv()
        cps.append(copy_out(comm_b, 1, right * m_per, 3))

        diag = (my_pos + 2) % N_DEV
        f2.wait_recv()
        cps.append(copy_out(comm_f, 2, diag * m_per, 4))
        b2.wait_recv()
        cps.append(copy_out(comm_b, 2, diag * m_per + half, 5))

        own.wait()
        for cp in cps:
            cp.wait()
        for r in [f0, b0, f1, b1, f2, b2]:
            r.wait_send()

    x = pltpu.with_memory_space_constraint(x, pltpu.MemorySpace.HBM)
    return pl.pallas_call(
        body,
        out_shape=jax.ShapeDtypeStruct((N_DEV * m_per, n), x.dtype),
        in_specs=[pl.BlockSpec(memory_space=pltpu.MemorySpace.HBM)],
        out_specs=pl.BlockSpec(memory_space=pltpu.MemorySpace.HBM),
        scratch_shapes=[
            pltpu.VMEM((3, half, n), x.dtype),
            pltpu.VMEM((3, half, n), x.dtype),
            pltpu.SemaphoreType.DMA((6,)),
            pltpu.SemaphoreType.DMA((6,)),
            pltpu.SemaphoreType.DMA((6,)),
            pltpu.SemaphoreType.DMA,
        ],
        compiler_params=pltpu.CompilerParams(collective_id=0),
    )(x)


# device time: 78685 ns/iter; 1.0653x vs baseline; 1.0041x over previous
import jax
import jax.numpy as jnp
from jax import lax
from jax.experimental import pallas as pl
from jax.experimental.pallas import tpu as pltpu

N_DEV = 4


def kernel(x):
    m_per, n = x.shape
    half = m_per // 2

    def body(x_ref, out_ref, s_sems, r_sems, own_sem):
        my_pos = lax.axis_index("i")
        left = (my_pos - 1) % N_DEV
        right = (my_pos + 1) % N_DEV

        barrier_sem = pltpu.get_barrier_semaphore()
        for nbr in [left, right]:
            pl.semaphore_signal(
                barrier_sem, inc=1,
                device_id=(nbr,), device_id_type=pl.DeviceIdType.MESH,
            )
        pl.semaphore_wait(barrier_sem, 2)

        def remote(src, dst_sl, target, i):
            return pltpu.make_async_remote_copy(
                src_ref=src,
                dst_ref=out_ref.at[dst_sl],
                send_sem=s_sems.at[i],
                recv_sem=r_sems.at[i],
                device_id=(target,),
                device_id_type=pl.DeviceIdType.MESH,
            )

        top = pl.ds(my_pos * m_per, half)
        bot = pl.ds(my_pos * m_per + half, half)
        f0 = remote(x_ref.at[pl.ds(0, half)], top, right, 0)
        b0 = remote(x_ref.at[pl.ds(half, half)], bot, left, 1)
        f1 = remote(x_ref.at[pl.ds(half, half)], bot, right, 2)
        b1 = remote(x_ref.at[pl.ds(0, half)], top, left, 3)
        f0.start()
        b0.start()
        f1.start()
        b1.start()

        own = pltpu.make_async_copy(
            x_ref, out_ref.at[pl.ds(my_pos * m_per, m_per)], own_sem
        )
        own.start()

        l_top = pl.ds(left * m_per, half)
        r_bot = pl.ds(right * m_per + half, half)
        f0.wait_recv()
        f2 = remote(out_ref.at[l_top], l_top, right, 4)
        f2.start()
        b0.wait_recv()
        b2 = remote(out_ref.at[r_bot], r_bot, left, 5)
        b2.start()

        f1.wait_recv()
        b1.wait_recv()
        f2.wait_recv()
        b2.wait_recv()

        own.wait()
        for r in [f0, b0, f1, b1, f2, b2]:
            r.wait_send()

    x = pltpu.with_memory_space_constraint(x, pltpu.MemorySpace.HBM)
    return pl.pallas_call(
        body,
        out_shape=jax.ShapeDtypeStruct((N_DEV * m_per, n), x.dtype),
        in_specs=[pl.BlockSpec(memory_space=pltpu.MemorySpace.HBM)],
        out_specs=pl.BlockSpec(memory_space=pltpu.MemorySpace.HBM),
        scratch_shapes=[
            pltpu.SemaphoreType.DMA((6,)),
            pltpu.SemaphoreType.DMA((6,)),
            pltpu.SemaphoreType.DMA,
        ],
        compiler_params=pltpu.CompilerParams(collective_id=0),
    )(x)
